# baseline (speedup 1.0000x reference)
import functools

import jax
import jax.numpy as jnp
from jax import lax
from jax.experimental import pallas as pl
from jax.experimental.pallas import tpu as pltpu
from jax.experimental.pallas import tpu_sc as plsc

NUM_USERS = 5000
N_NODES = 10000
N_EDGES = 320000
EMB = 128
HALF = EMB // 2
N_LAYERS = 3

NC = 2
NS = 16
LANES = 16

CHUNK = 128
NHALF = 4
HCHUNKS = 40
E_PAD = NS * NHALF * HCHUNKS * CHUNK
STRIPE = 624
ZCHUNK = 32


NPAR = 4
MAIN = HCHUNKS - HCHUNKS % NPAR


def _sc_body(col_hbm, row_hbm, val_hbm, ego0_hbm, out1, out2, out3,
             col_v, row_v, val_v, gbufs, sbufs, zero_v, accum,
             gsems, ssems):
    c = lax.axis_index("c")
    s = lax.axis_index("s")

    off = (c * N_NODES).astype(jnp.int32)

    @pl.loop(0, ZCHUNK)
    def _mkzero(j):
        for t in range(HALF // LANES):
            zero_v[j, pl.ds(t * LANES, LANES)] = jnp.zeros((LANES,), jnp.float32)

    for src, dst in ((ego0_hbm, out1), (out1, out2), (out2, out3)):
        for z in range(640 // ZCHUNK):
            pltpu.sync_copy(zero_v, accum.at[pl.ds(s * STRIPE + z * ZCHUNK, ZCHUNK)])
        plsc.subcore_barrier()

        @pl.loop(0, NHALF)
        def _half(h):
            pltpu.sync_copy(col_hbm.at[s, h], col_v)
            pltpu.sync_copy(row_hbm.at[s, h], row_v)
            pltpu.sync_copy(val_hbm.at[s, h], val_v)

            @pl.loop(0, HCHUNKS)
            def _add_off(j):
                for t in range(CHUNK // LANES):
                    sl = (j, pl.ds(t * LANES, LANES))
                    col_v[sl] = col_v[sl] + off

            def _stage(kk, par, guarded):
                gb, sb = gbufs[par], sbufs[par]
                gsem, ssem = gsems[par], ssems[par]
                pltpu.make_async_copy(src.at[col_v.at[kk]], gb, gsem).wait()

                def _drain():
                    pltpu.make_async_copy(sb, accum.at[row_v.at[kk]], ssem).wait()

                if guarded:
                    pl.when(kk >= NPAR)(_drain)
                else:
                    _drain()

                @pl.loop(0, CHUNK // LANES)
                def _scale(g):
                    vv16 = val_v[kk, pl.ds(g * LANES, LANES)]
                    for j in range(LANES):
                        vv = jnp.full((LANES,), vv16[j], jnp.float32)
                        for t in range(HALF // LANES):
                            sl = (g * LANES + j, pl.ds(t * LANES, LANES))
                            sb[sl] = gb[sl] * vv

                def _prefetch():
                    pltpu.async_copy(src.at[col_v.at[kk + NPAR]], gb, gsem)

                if guarded:
                    pl.when(kk + NPAR < HCHUNKS)(_prefetch)
                elif kk + NPAR < HCHUNKS:
                    _prefetch()

                pltpu.async_copy(sb, accum.at[row_v.at[kk]], ssem, add=True)

            for p in range(NPAR):
                pltpu.async_copy(src.at[col_v.at[p]], gbufs[p], gsems[p])

            @pl.loop(0, MAIN, step=NPAR)
            def _chunk(k):
                for par in range(NPAR):
                    _stage(k + par, par, guarded=True)

            for kk in range(MAIN, HCHUNKS):
                _stage(kk, kk % NPAR, guarded=False)

            for p in range(NPAR):
                pltpu.make_async_copy(sbufs[p], accum.at[row_v.at[0]], ssems[p]).wait()

        plsc.subcore_barrier()
        pltpu.sync_copy(accum.at[pl.ds(s * STRIPE, STRIPE)],
                        dst.at[pl.ds(c * N_NODES + s * STRIPE, STRIPE)])

        @pl.when(s == NS - 1)
        def _tail():
            pltpu.sync_copy(accum.at[pl.ds(NS * STRIPE, N_NODES - NS * STRIPE)],
                            dst.at[pl.ds(c * N_NODES + NS * STRIPE, N_NODES - NS * STRIPE)])

        plsc.subcore_barrier()


_sc_spmm3 = pl.kernel(
    _sc_body,
    out_type=[jax.ShapeDtypeStruct((NC * N_NODES, HALF), jnp.float32)] * N_LAYERS,
    mesh=plsc.VectorSubcoreMesh(core_axis_name="c", subcore_axis_name="s",
                                num_cores=NC, num_subcores=NS),
    scratch_types=[
        pltpu.VMEM((HCHUNKS, CHUNK), jnp.int32),
        pltpu.VMEM((HCHUNKS, CHUNK), jnp.int32),
        pltpu.VMEM((HCHUNKS, CHUNK), jnp.float32),
        [pltpu.VMEM((CHUNK, HALF), jnp.float32)] * NPAR,
        [pltpu.VMEM((CHUNK, HALF), jnp.float32)] * NPAR,
        pltpu.VMEM((ZCHUNK, HALF), jnp.float32),
        pltpu.VMEM_SHARED((N_NODES, HALF), jnp.float32),
        [pltpu.SemaphoreType.DMA] * NPAR,
        [pltpu.SemaphoreType.DMA] * NPAR,
    ],
    compiler_params=pltpu.CompilerParams(use_tc_tiling_on_sc=False),
)


def _mean_body(e0l, e1l, e2l, e3l, e0r, e1r, e2r, e3r, o):
    left = (e0l[...] + e1l[...] + e2l[...] + e3l[...]) * 0.25
    right = (e0r[...] + e1r[...] + e2r[...] + e3r[...]) * 0.25
    o[...] = jnp.concatenate([left, right], axis=1)


_tc_mean_call = pl.pallas_call(
    _mean_body,
    grid=(10,),
    in_specs=[pl.BlockSpec((N_NODES // 10, HALF),
                           functools.partial(lambda h, i: (h * 10 + i, 0), h))
              for h in range(NC) for _ in range(4)],
    out_specs=pl.BlockSpec((N_NODES // 10, EMB), lambda i: (i, 0)),
    out_shape=jax.ShapeDtypeStruct((N_NODES, EMB), jnp.float32),
)


def _tc_mean(e0, e1, e2, e3):
    return _tc_mean_call(e0, e1, e2, e3, e0, e1, e2, e3)


def kernel(embedding_user, embedding_item, adj_vals, adj_row, adj_col):
    ego0 = jnp.concatenate([embedding_user, embedding_item], axis=0)
    ego0_s = jnp.concatenate([ego0[:, :HALF], ego0[:, HALF:]], axis=0)
    pad = E_PAD - N_EDGES
    shape4 = (NS, NHALF, HCHUNKS, CHUNK)
    spread = (jnp.arange(pad, dtype=jnp.int32) * 13) % N_NODES
    col2 = jnp.concatenate([adj_col, spread]).reshape(shape4)
    row2 = jnp.concatenate([adj_row, spread]).reshape(shape4)
    val2 = jnp.concatenate(
        [adj_vals, jnp.zeros((pad,), jnp.float32)]).reshape(shape4)
    ego1_s, ego2_s, ego3_s = _sc_spmm3(col2, row2, val2, ego0_s)
    mean = _tc_mean(ego0_s, ego1_s, ego2_s, ego3_s)
    return mean[:NUM_USERS], mean[NUM_USERS:]


# speedup vs baseline: 1.0816x; 1.0816x over previous
import functools

import jax
import jax.numpy as jnp
from jax import lax
from jax.experimental import pallas as pl
from jax.experimental.pallas import tpu as pltpu
from jax.experimental.pallas import tpu_sc as plsc

NUM_USERS = 5000
N_NODES = 10000
N_EDGES = 320000
EMB = 128
HALF = EMB // 2
N_LAYERS = 3

NC = 2
NS = 16
LANES = 16

CHUNK = 80
CHUNKS_TOTAL = N_EDGES // CHUNK
CHUNKS_PER_TILE = CHUNKS_TOTAL // NS
NHALF = 2
HCHUNKS = CHUNKS_PER_TILE // NHALF
STRIPE = 624
ZCHUNK = 32


NPAR = 4
MAIN = HCHUNKS - HCHUNKS % NPAR


def _sc_body(col_hbm, row_hbm, val_hbm, ego0_hbm, out1, out2, out3,
             col_v, row_v, val_v, gbufs, sbufs, zero_v, accum,
             gsems, ssems):
    c = lax.axis_index("c")
    s = lax.axis_index("s")

    off = (c * N_NODES).astype(jnp.int32)

    @pl.loop(0, ZCHUNK)
    def _mkzero(j):
        for t in range(HALF // LANES):
            zero_v[j, pl.ds(t * LANES, LANES)] = jnp.zeros((LANES,), jnp.float32)

    for src, dst in ((ego0_hbm, out1), (out1, out2), (out2, out3)):
        for z in range(640 // ZCHUNK):
            pltpu.sync_copy(zero_v, accum.at[pl.ds(s * STRIPE + z * ZCHUNK, ZCHUNK)])
        plsc.subcore_barrier()

        @pl.loop(0, NHALF)
        def _half(h):
            pltpu.sync_copy(col_hbm.at[s, h], col_v)
            pltpu.sync_copy(row_hbm.at[s, h], row_v)
            pltpu.sync_copy(val_hbm.at[s, h], val_v)

            @pl.loop(0, HCHUNKS)
            def _add_off(j):
                for t in range(CHUNK // LANES):
                    sl = (j, pl.ds(t * LANES, LANES))
                    col_v[sl] = col_v[sl] + off

            def _stage(kk, par, guarded):
                gb, sb = gbufs[par], sbufs[par]
                gsem, ssem = gsems[par], ssems[par]
                pltpu.make_async_copy(src.at[col_v.at[kk]], gb, gsem).wait()

                def _drain():
                    pltpu.make_async_copy(sb, accum.at[row_v.at[kk]], ssem).wait()

                if guarded:
                    pl.when(kk >= NPAR)(_drain)
                else:
                    _drain()

                @pl.loop(0, CHUNK // LANES)
                def _scale(g):
                    vv16 = val_v[kk, pl.ds(g * LANES, LANES)]
                    for j in range(LANES):
                        idx = jnp.full((LANES,), j, jnp.int32)
                        vv = vv16.at[idx].get(mode="promise_in_bounds")
                        for t in range(HALF // LANES):
                            sl = (g * LANES + j, pl.ds(t * LANES, LANES))
                            sb[sl] = gb[sl] * vv

                def _prefetch():
                    pltpu.async_copy(src.at[col_v.at[kk + NPAR]], gb, gsem)

                if guarded:
                    pl.when(kk + NPAR < HCHUNKS)(_prefetch)
                elif kk + NPAR < HCHUNKS:
                    _prefetch()

                pltpu.async_copy(sb, accum.at[row_v.at[kk]], ssem, add=True)

            for p in range(NPAR):
                pltpu.async_copy(src.at[col_v.at[p]], gbufs[p], gsems[p])

            @pl.loop(0, MAIN, step=NPAR)
            def _chunk(k):
                for par in range(NPAR):
                    _stage(k + par, par, guarded=True)

            for kk in range(MAIN, HCHUNKS):
                _stage(kk, kk % NPAR, guarded=False)

            for p in range(NPAR):
                pltpu.make_async_copy(sbufs[p], accum.at[row_v.at[0]], ssems[p]).wait()

        plsc.subcore_barrier()
        pltpu.sync_copy(accum.at[pl.ds(s * STRIPE, STRIPE)],
                        dst.at[pl.ds(c * N_NODES + s * STRIPE, STRIPE)])

        @pl.when(s == NS - 1)
        def _tail():
            pltpu.sync_copy(accum.at[pl.ds(NS * STRIPE, N_NODES - NS * STRIPE)],
                            dst.at[pl.ds(c * N_NODES + NS * STRIPE, N_NODES - NS * STRIPE)])

        plsc.subcore_barrier()


_sc_spmm3 = pl.kernel(
    _sc_body,
    out_type=[jax.ShapeDtypeStruct((NC * N_NODES, HALF), jnp.float32)] * N_LAYERS,
    mesh=plsc.VectorSubcoreMesh(core_axis_name="c", subcore_axis_name="s",
                                num_cores=NC, num_subcores=NS),
    scratch_types=[
        pltpu.VMEM((HCHUNKS, CHUNK), jnp.int32),
        pltpu.VMEM((HCHUNKS, CHUNK), jnp.int32),
        pltpu.VMEM((HCHUNKS, CHUNK), jnp.float32),
        [pltpu.VMEM((CHUNK, HALF), jnp.float32)] * NPAR,
        [pltpu.VMEM((CHUNK, HALF), jnp.float32)] * NPAR,
        pltpu.VMEM((ZCHUNK, HALF), jnp.float32),
        pltpu.VMEM_SHARED((N_NODES, HALF), jnp.float32),
        [pltpu.SemaphoreType.DMA] * NPAR,
        [pltpu.SemaphoreType.DMA] * NPAR,
    ],
    compiler_params=pltpu.CompilerParams(use_tc_tiling_on_sc=False),
)


def _mean_body(e0l, e1l, e2l, e3l, e0r, e1r, e2r, e3r, o):
    left = (e0l[...] + e1l[...] + e2l[...] + e3l[...]) * 0.25
    right = (e0r[...] + e1r[...] + e2r[...] + e3r[...]) * 0.25
    o[...] = jnp.concatenate([left, right], axis=1)


_tc_mean_call = pl.pallas_call(
    _mean_body,
    grid=(10,),
    in_specs=[pl.BlockSpec((N_NODES // 10, HALF),
                           functools.partial(lambda h, i: (h * 10 + i, 0), h))
              for h in range(NC) for _ in range(4)],
    out_specs=pl.BlockSpec((N_NODES // 10, EMB), lambda i: (i, 0)),
    out_shape=jax.ShapeDtypeStruct((N_NODES, EMB), jnp.float32),
)


def _tc_mean(e0, e1, e2, e3):
    return _tc_mean_call(e0, e1, e2, e3, e0, e1, e2, e3)


def kernel(embedding_user, embedding_item, adj_vals, adj_row, adj_col):
    ego0 = jnp.concatenate([embedding_user, embedding_item], axis=0)
    ego0_s = jnp.concatenate([ego0[:, :HALF], ego0[:, HALF:]], axis=0)
    col2 = adj_col.reshape(NS, NHALF, HCHUNKS, CHUNK)
    row2 = adj_row.reshape(NS, NHALF, HCHUNKS, CHUNK)
    val2 = adj_vals.reshape(NS, NHALF, HCHUNKS, CHUNK)
    ego1_s, ego2_s, ego3_s = _sc_spmm3(col2, row2, val2, ego0_s)
    mean = _tc_mean(ego0_s, ego1_s, ego2_s, ego3_s)
    return mean[:NUM_USERS], mean[NUM_USERS:]


# speedup vs baseline: 1.1025x; 1.0193x over previous
import functools

import jax
import jax.numpy as jnp
from jax import lax
from jax.experimental import pallas as pl
from jax.experimental.pallas import tpu as pltpu
from jax.experimental.pallas import tpu_sc as plsc

NUM_USERS = 5000
N_NODES = 10000
N_EDGES = 320000
EMB = 128
HALF = EMB // 2
N_LAYERS = 3

NC = 2
NS = 16
LANES = 16

CHUNK = 80
CHUNKS_TOTAL = N_EDGES // CHUNK
CHUNKS_PER_TILE = CHUNKS_TOTAL // NS
NHALF = 2
HCHUNKS = CHUNKS_PER_TILE // NHALF
STRIPE = 624
ZCHUNK = 48


NPAR = 4
MAIN = HCHUNKS - HCHUNKS % NPAR


def _sc_body(col_hbm, row_hbm, val_hbm, ego0_hbm, out1, out2, out3,
             col_v, row_v, val_v, gbufs, sbufs, zero_v, accum,
             gsems, ssems):
    c = lax.axis_index("c")
    s = lax.axis_index("s")

    off = (c * N_NODES).astype(jnp.int32)

    @pl.loop(0, ZCHUNK)
    def _mkzero(j):
        for t in range(HALF // LANES):
            zero_v[j, pl.ds(t * LANES, LANES)] = jnp.zeros((LANES,), jnp.float32)

    def _zero_stripe():
        for z in range(STRIPE // ZCHUNK):
            pltpu.sync_copy(zero_v, accum.at[pl.ds(s * STRIPE + z * ZCHUNK, ZCHUNK)])

        @pl.when(s == NS - 1)
        def _ztail():
            pltpu.sync_copy(zero_v.at[pl.ds(0, N_NODES - NS * STRIPE)],
                            accum.at[pl.ds(NS * STRIPE, N_NODES - NS * STRIPE)])

    _zero_stripe()
    plsc.subcore_barrier()

    for src, dst in ((ego0_hbm, out1), (out1, out2), (out2, out3)):

        @pl.loop(0, NHALF)
        def _half(h):
            stg = [pltpu.async_copy(col_hbm.at[s, h], col_v, gsems[0]),
                   pltpu.async_copy(row_hbm.at[s, h], row_v, gsems[1]),
                   pltpu.async_copy(val_hbm.at[s, h], val_v, gsems[2])]
            for d in stg:
                d.wait()

            @pl.loop(0, HCHUNKS)
            def _add_off(j):
                for t in range(CHUNK // LANES):
                    sl = (j, pl.ds(t * LANES, LANES))
                    col_v[sl] = col_v[sl] + off

            def _stage(kk, par, guarded):
                gb, sb = gbufs[par], sbufs[par]
                gsem, ssem = gsems[par], ssems[par]
                pltpu.make_async_copy(src.at[col_v.at[kk]], gb, gsem).wait()

                def _drain():
                    pltpu.make_async_copy(sb, accum.at[row_v.at[kk]], ssem).wait()

                if guarded:
                    pl.when(kk >= NPAR)(_drain)
                else:
                    _drain()

                @pl.loop(0, CHUNK // LANES)
                def _scale(g):
                    vv16 = val_v[kk, pl.ds(g * LANES, LANES)]
                    for j in range(LANES):
                        idx = jnp.full((LANES,), j, jnp.int32)
                        vv = vv16.at[idx].get(mode="promise_in_bounds")
                        for t in range(HALF // LANES):
                            sl = (g * LANES + j, pl.ds(t * LANES, LANES))
                            sb[sl] = gb[sl] * vv

                def _prefetch():
                    pltpu.async_copy(src.at[col_v.at[kk + NPAR]], gb, gsem)

                if guarded:
                    pl.when(kk + NPAR < HCHUNKS)(_prefetch)
                elif kk + NPAR < HCHUNKS:
                    _prefetch()

                pltpu.async_copy(sb, accum.at[row_v.at[kk]], ssem, add=True)

            for p in range(NPAR):
                pltpu.async_copy(src.at[col_v.at[p]], gbufs[p], gsems[p])

            @pl.loop(0, MAIN, step=NPAR)
            def _chunk(k):
                for par in range(NPAR):
                    _stage(k + par, par, guarded=True)

            for kk in range(MAIN, HCHUNKS):
                _stage(kk, kk % NPAR, guarded=False)

            for p in range(NPAR):
                pltpu.make_async_copy(sbufs[p], accum.at[row_v.at[0]], ssems[p]).wait()

        plsc.subcore_barrier()
        pltpu.sync_copy(accum.at[pl.ds(s * STRIPE, STRIPE)],
                        dst.at[pl.ds(c * N_NODES + s * STRIPE, STRIPE)])

        @pl.when(s == NS - 1)
        def _tail():
            pltpu.sync_copy(accum.at[pl.ds(NS * STRIPE, N_NODES - NS * STRIPE)],
                            dst.at[pl.ds(c * N_NODES + NS * STRIPE, N_NODES - NS * STRIPE)])

        if dst is not out3:
            _zero_stripe()
        plsc.subcore_barrier()


_sc_spmm3 = pl.kernel(
    _sc_body,
    out_type=[jax.ShapeDtypeStruct((NC * N_NODES, HALF), jnp.float32)] * N_LAYERS,
    mesh=plsc.VectorSubcoreMesh(core_axis_name="c", subcore_axis_name="s",
                                num_cores=NC, num_subcores=NS),
    scratch_types=[
        pltpu.VMEM((HCHUNKS, CHUNK), jnp.int32),
        pltpu.VMEM((HCHUNKS, CHUNK), jnp.int32),
        pltpu.VMEM((HCHUNKS, CHUNK), jnp.float32),
        [pltpu.VMEM((CHUNK, HALF), jnp.float32)] * NPAR,
        [pltpu.VMEM((CHUNK, HALF), jnp.float32)] * NPAR,
        pltpu.VMEM((ZCHUNK, HALF), jnp.float32),
        pltpu.VMEM_SHARED((N_NODES, HALF), jnp.float32),
        [pltpu.SemaphoreType.DMA] * NPAR,
        [pltpu.SemaphoreType.DMA] * NPAR,
    ],
    compiler_params=pltpu.CompilerParams(use_tc_tiling_on_sc=False),
)


def _mean_body(e0l, e1l, e2l, e3l, e0r, e1r, e2r, e3r, o):
    left = (e0l[...] + e1l[...] + e2l[...] + e3l[...]) * 0.25
    right = (e0r[...] + e1r[...] + e2r[...] + e3r[...]) * 0.25
    o[...] = jnp.concatenate([left, right], axis=1)


_tc_mean_call = pl.pallas_call(
    _mean_body,
    grid=(10,),
    in_specs=[pl.BlockSpec((N_NODES // 10, HALF),
                           functools.partial(lambda h, i: (h * 10 + i, 0), h))
              for h in range(NC) for _ in range(4)],
    out_specs=pl.BlockSpec((N_NODES // 10, EMB), lambda i: (i, 0)),
    out_shape=jax.ShapeDtypeStruct((N_NODES, EMB), jnp.float32),
)


def _tc_mean(e0, e1, e2, e3):
    return _tc_mean_call(e0, e1, e2, e3, e0, e1, e2, e3)


def kernel(embedding_user, embedding_item, adj_vals, adj_row, adj_col):
    ego0 = jnp.concatenate([embedding_user, embedding_item], axis=0)
    ego0_s = jnp.concatenate([ego0[:, :HALF], ego0[:, HALF:]], axis=0)
    col2 = adj_col.reshape(NS, NHALF, HCHUNKS, CHUNK)
    row2 = adj_row.reshape(NS, NHALF, HCHUNKS, CHUNK)
    val2 = adj_vals.reshape(NS, NHALF, HCHUNKS, CHUNK)
    ego1_s, ego2_s, ego3_s = _sc_spmm3(col2, row2, val2, ego0_s)
    mean = _tc_mean(ego0_s, ego1_s, ego2_s, ego3_s)
    return mean[:NUM_USERS], mean[NUM_USERS:]
